# baseline (speedup 1.0000x reference)
import functools

import jax
import jax.numpy as jnp
from jax import lax
from jax.experimental import pallas as pl
from jax.experimental.pallas import tpu as pltpu
from jax.experimental.pallas import tpu_sc as plsc

_DIM = 32
_L = 16
_CHUNK = 128


def _mod1(v):
    r = lax.rem(v, 1.0)
    return jnp.where(r < 0.0, r + 1.0, r)


@functools.lru_cache(maxsize=None)
def _make_sc_kernel(n_idx, nw, nc, n_chunks, chunk):
    per_w = n_idx // nw
    mesh = plsc.VectorSubcoreMesh(core_axis_name="c", subcore_axis_name="s")
    out_rows = jax.ShapeDtypeStruct((n_idx, _DIM), jnp.float32)

    @functools.partial(
        pl.kernel,
        mesh=mesh,
        out_type=(out_rows,) * 4,
        compiler_params=pltpu.CompilerParams(use_tc_tiling_on_sc=False),
        scratch_types=[
            pltpu.VMEM((n_chunks, chunk), jnp.int32),
            pltpu.VMEM((chunk, _DIM), jnp.float32),
            pltpu.VMEM((chunk, _DIM), jnp.float32),
            pltpu.VMEM((chunk, _DIM), jnp.float32),
            pltpu.VMEM((chunk, _DIM), jnp.float32),
            pltpu.SemaphoreType.DMA,
            pltpu.SemaphoreType.DMA,
        ],
    )
    def sc_kernel(x_hbm, bt_hbm, et_hbm, ob, oe, od, oa,
                  idx_v, brow, erow, dbuf, abuf, semb, seme):
        c = lax.axis_index("c")
        s = lax.axis_index("s")
        wid = s * nc + c
        base = wid * per_w
        pltpu.sync_copy(x_hbm.at[wid], idx_v)

        def step(j, carry):
            cb = pltpu.async_copy(bt_hbm.at[idx_v.at[j]], brow, semb)
            ce = pltpu.async_copy(et_hbm.at[idx_v.at[j]], erow, seme)
            cb.wait()
            ce.wait()

            def row(r, carry2):
                for h in range(_DIM // _L):
                    sl = pl.ds(h * _L, _L)
                    bseg = brow[r, sl]
                    eseg = erow[r, sl]
                    dbuf[r, sl] = _mod1(bseg + eseg)
                    abuf[r, sl] = _mod1(bseg + 2.0 * eseg)
                return carry2

            lax.fori_loop(0, chunk, row, 0)
            row0 = base + j * chunk
            pltpu.sync_copy(brow, ob.at[pl.ds(row0, chunk)])
            pltpu.sync_copy(erow, oe.at[pl.ds(row0, chunk)])
            pltpu.sync_copy(dbuf, od.at[pl.ds(row0, chunk)])
            pltpu.sync_copy(abuf, oa.at[pl.ds(row0, chunk)])
            return carry

        lax.fori_loop(0, n_chunks, step, 0)

    return sc_kernel


def kernel(x, b_table, e_table):
    info = plsc.get_sparse_core_info()
    nw = info.num_cores * info.num_subcores
    nc = info.num_cores
    n_idx = x.size
    per_w = n_idx // nw
    n_chunks = per_w // _CHUNK
    xf = x.reshape(-1).astype(jnp.int32).reshape(nw, n_chunks, _CHUNK)
    k = _make_sc_kernel(n_idx, nw, nc, n_chunks, _CHUNK)
    ob, oe, od, oa = k(xf, b_table, e_table)
    shp = x.shape + (_DIM,)
    return (ob.reshape(shp), oe.reshape(shp), od.reshape(shp),
            oa.reshape(shp))


# speedup vs baseline: 1.0810x; 1.0810x over previous
import functools

import jax
import jax.numpy as jnp
from jax import lax
from jax.experimental import pallas as pl
from jax.experimental.pallas import tpu as pltpu
from jax.experimental.pallas import tpu_sc as plsc

_DIM = 32
_L = 16
_CHUNK = 128
_NB = 4
_PF = 2
_UR = 4


def _mod1(v):
    r = lax.rem(v, 1.0)
    return jnp.where(r < 0.0, r + 1.0, r)


@functools.lru_cache(maxsize=None)
def _make_sc_kernel(n_idx, nw, nc, n_chunks, chunk):
    per_w = n_idx // nw
    assert n_chunks % _NB == 0
    mesh = plsc.VectorSubcoreMesh(core_axis_name="c", subcore_axis_name="s")
    out_rows = jax.ShapeDtypeStruct((n_idx, _DIM), jnp.float32)

    @functools.partial(
        pl.kernel,
        mesh=mesh,
        out_type=(out_rows,) * 4,
        compiler_params=pltpu.CompilerParams(use_tc_tiling_on_sc=False),
        scratch_types=[
            pltpu.VMEM((n_chunks, chunk), jnp.int32),
            pltpu.VMEM((_NB, chunk, _DIM), jnp.float32),
            pltpu.VMEM((_NB, chunk, _DIM), jnp.float32),
            pltpu.VMEM((_NB, chunk, _DIM), jnp.float32),
            pltpu.VMEM((_NB, chunk, _DIM), jnp.float32),
        ] + [pltpu.SemaphoreType.DMA] * (2 * _NB),
    )
    def sc_kernel(x_hbm, bt, et, ob, oe, od, oa,
                  idx_v, brow, erow, dbuf, abuf, *sems):
        gsem = sems[:_NB]
        wsem = sems[_NB:]
        c = lax.axis_index("c")
        s = lax.axis_index("s")
        wid = s * nc + c
        base = wid * per_w
        pltpu.sync_copy(x_hbm.at[wid], idx_v)

        def issue_gather(j, slot):
            pltpu.async_copy(bt.at[idx_v.at[j]], brow.at[slot], gsem[slot])
            pltpu.async_copy(et.at[idx_v.at[j]], erow.at[slot], gsem[slot])

        for b in range(_PF):
            issue_gather(b, b)

        def group(gi, carry):
            for b in range(_NB):
                j = gi * _NB + b
                pltpu.make_async_copy(
                    bt.at[idx_v.at[j]], brow.at[b], gsem[b]).wait()
                pltpu.make_async_copy(
                    et.at[idx_v.at[j]], erow.at[b], gsem[b]).wait()

                def rows(r4, c2, _b=b):
                    for u in range(_UR):
                        r = r4 * _UR + u
                        for h in range(_DIM // _L):
                            sl = pl.ds(h * _L, _L)
                            bv = brow[_b, r, sl]
                            ev = erow[_b, r, sl]
                            dbuf[_b, r, sl] = _mod1(bv + ev)
                            abuf[_b, r, sl] = _mod1(bv + 2.0 * ev)
                    return c2

                lax.fori_loop(0, chunk // _UR, rows, 0)

                row0 = base + j * chunk
                pltpu.async_copy(brow.at[b], ob.at[pl.ds(row0, chunk)], wsem[b])
                pltpu.async_copy(erow.at[b], oe.at[pl.ds(row0, chunk)], wsem[b])
                pltpu.async_copy(dbuf.at[b], od.at[pl.ds(row0, chunk)], wsem[b])
                pltpu.async_copy(abuf.at[b], oa.at[pl.ds(row0, chunk)], wsem[b])

                s2 = (b + _PF) % _NB

                @pl.when(j >= _NB - _PF)
                def _drain(_s2=s2):
                    for _ in range(4):
                        pltpu.make_async_copy(
                            dbuf.at[_s2], od.at[pl.ds(0, chunk)],
                            wsem[_s2]).wait()

                @pl.when(j + _PF < n_chunks)
                def _prefetch(_j=j, _s2=s2):
                    issue_gather(_j + _PF, _s2)
            return carry

        lax.fori_loop(0, n_chunks // _NB, group, 0)

        for jj in range(n_chunks - (_NB - _PF), n_chunks):
            slot = jj % _NB
            for _ in range(4):
                pltpu.make_async_copy(
                    dbuf.at[slot], od.at[pl.ds(0, chunk)], wsem[slot]).wait()

    return sc_kernel


def kernel(x, b_table, e_table):
    info = plsc.get_sparse_core_info()
    nw = info.num_cores * info.num_subcores
    nc = info.num_cores
    n_idx = x.size
    per_w = n_idx // nw
    n_chunks = per_w // _CHUNK
    xf = x.reshape(-1).astype(jnp.int32).reshape(nw, n_chunks, _CHUNK)
    k = _make_sc_kernel(n_idx, nw, nc, n_chunks, _CHUNK)
    ob, oe, od, oa = k(xf, b_table, e_table)
    shp = x.shape + (_DIM,)
    return (ob.reshape(shp), oe.reshape(shp), od.reshape(shp),
            oa.reshape(shp))


# speedup vs baseline: 2.0893x; 1.9327x over previous
import functools

import jax
import jax.numpy as jnp
from jax import lax
from jax.experimental import pallas as pl
from jax.experimental.pallas import tpu as pltpu
from jax.experimental.pallas import tpu_sc as plsc

_DIM = 32
_L = 16
_CHUNK = 128
_RING = 4
_PF = 3


def _mod1(v):
    r = lax.rem(v, 1.0)
    return jnp.where(r < 0.0, r + 1.0, r)


@functools.lru_cache(maxsize=None)
def _make_sc_kernel(n_j, n_i, nw, nc):
    ipw = n_i // nw
    tcw = ipw // _CHUNK
    n_chunks = n_j * tcw
    n_groups = n_chunks // _RING
    n_ktiles = _DIM // 8
    n_tc = n_i // _CHUNK
    out_flat = jax.ShapeDtypeStruct((n_j * n_ktiles * n_tc * 8, _CHUNK),
                                    jnp.float32)
    mesh = plsc.VectorSubcoreMesh(core_axis_name="c", subcore_axis_name="s")

    @functools.partial(
        pl.kernel,
        mesh=mesh,
        out_type=(out_flat,) * 4,
        compiler_params=pltpu.CompilerParams(
            use_tc_tiling_on_sc=False, needs_layout_passes=False),
        scratch_types=(
            [pltpu.VMEM((n_chunks, _CHUNK), jnp.int32)]
            + [pltpu.VMEM((_CHUNK, _DIM), jnp.float32)] * (2 * _RING)
            + [pltpu.VMEM((_DIM, _CHUNK), jnp.float32)] * 8
            + [pltpu.SemaphoreType.DMA] * (_RING + 2)
        ),
    )
    def sc_kernel(xq, bt, et, ob, oe, od, oa, idx_v,
                  b0, b1, b2, b3, e0, e1, e2, e3,
                  sb0, se0, sd0, sa0, sb1, se1, sd1, sa1,
                  g0, g1, g2, g3, w0, w1):
        brefs = (b0, b1, b2, b3)
        erefs = (e0, e1, e2, e3)
        gsem = (g0, g1, g2, g3)
        wsem = (w0, w1)
        stag = ((sb0, se0, sd0, sa0), (sb1, se1, sd1, sa1))
        c_ax = lax.axis_index("c")
        s_ax = lax.axis_index("s")
        wid = s_ax * nc + c_ax
        pltpu.sync_copy(xq.at[wid], idx_v)

        ivs = [lax.iota(jnp.int32, 16) + (lg * _L) for lg in range(8)]

        def issue_gather(cidx, slot):
            pltpu.async_copy(bt.at[idx_v.at[cidx]], brefs[slot], gsem[slot])
            pltpu.async_copy(et.at[idx_v.at[cidx]], erefs[slot], gsem[slot])

        for u in range(_PF):
            issue_gather(u, u)

        def group(gi, carry):
            for u in range(_RING):
                cc = gi * _RING + u
                s = u & 1
                sbs, ses, sds, sas = stag[s]
                bg = brefs[u]
                eg = erefs[u]

                pslot = (u + _PF) % _RING
                if u == 0:
                    issue_gather(cc + _PF, pslot)
                else:
                    @pl.when(gi < n_groups - 1)
                    def _pref(_cc=cc, _ps=pslot):
                        issue_gather(_cc + _PF, _ps)

                pltpu.make_async_copy(
                    bt.at[idx_v.at[cc]], bg, gsem[u]).wait()
                pltpu.make_async_copy(
                    et.at[idx_v.at[cc]], eg, gsem[u]).wait()

                def drain(_s=s):
                    for o_ref in stag[_s]:
                        for tr in range(n_ktiles):
                            pltpu.make_async_copy(
                                o_ref.at[pl.ds(tr * 8, 8)],
                                ob.at[pl.ds(0, 8)], wsem[_s]).wait()
                if u >= 2:
                    drain()
                else:
                    @pl.when(gi >= 1)
                    def _d():
                        drain()

                def kloop(kk, c2, _bg=bg, _eg=eg, _sbs=sbs, _ses=ses,
                          _sds=sds, _sas=sas):
                    kvec = jnp.full((16,), kk, dtype=jnp.int32)
                    for lg in range(8):
                        bv = plsc.load_gather(_bg, [ivs[lg], kvec])
                        ev = plsc.load_gather(_eg, [ivs[lg], kvec])
                        t = bv + ev
                        a0 = bv + 2.0 * ev
                        sl = pl.ds(lg * _L, _L)
                        _sbs[kk, sl] = bv
                        _ses[kk, sl] = ev
                        _sds[kk, sl] = _mod1(t)
                        _sas[kk, sl] = _mod1(a0)
                    return c2

                lax.fori_loop(0, _DIM, kloop, 0)

                j = cc // tcw
                t = cc % tcw
                for tr in range(n_ktiles):
                    q0 = ((j * n_ktiles + tr) * n_tc + wid * tcw + t) * 8
                    src = pl.ds(tr * 8, 8)
                    dst = pl.ds(q0, 8)
                    pltpu.async_copy(sbs.at[src], ob.at[dst], wsem[s])
                    pltpu.async_copy(ses.at[src], oe.at[dst], wsem[s])
                    pltpu.async_copy(sds.at[src], od.at[dst], wsem[s])
                    pltpu.async_copy(sas.at[src], oa.at[dst], wsem[s])
            return carry

        lax.fori_loop(0, n_groups, group, 0)

        for s in range(2):
            for o_ref in stag[s]:
                for tr in range(n_ktiles):
                    pltpu.make_async_copy(
                        o_ref.at[pl.ds(tr * 8, 8)],
                        ob.at[pl.ds(0, 8)], wsem[s]).wait()

    return sc_kernel


def kernel(x, b_table, e_table):
    info = plsc.get_sparse_core_info()
    nw = info.num_cores * info.num_subcores
    nc = info.num_cores
    n_i, n_j = x.shape
    ipw = n_i // nw
    tcw = ipw // _CHUNK
    xq = (x.T.astype(jnp.int32)
          .reshape(n_j, nw, tcw, _CHUNK)
          .transpose(1, 0, 2, 3)
          .reshape(nw, n_j * tcw, _CHUNK))
    k = _make_sc_kernel(n_j, n_i, nw, nc)
    outs = k(xq, b_table, e_table)
    n_ktiles = _DIM // 8
    n_tc = n_i // _CHUNK
    res = []
    for o in outs:
        v5 = o.reshape(n_j, n_ktiles, n_tc, 8, _CHUNK)
        res.append(v5.transpose(2, 4, 0, 1, 3).reshape(n_i, n_j, _DIM))
    return tuple(res)


# speedup vs baseline: 2.6044x; 1.2465x over previous
import functools

import jax
import jax.numpy as jnp
from jax import lax
from jax.experimental import pallas as pl
from jax.experimental.pallas import tpu as pltpu
from jax.experimental.pallas import tpu_sc as plsc

_DIM = 32
_L = 16
_CHUNK = 128
_RING = 4
_PF = 3
_PITCH = _CHUNK + 1


def _mod1(v):
    r = lax.rem(v, 1.0)
    return jnp.where(r < 0.0, r + 1.0, r)


@functools.lru_cache(maxsize=None)
def _make_sc_kernel(n_j, n_i, nw, nc):
    ipw = n_i // nw
    tcw = ipw // _CHUNK
    n_chunks = n_j * tcw
    n_groups = n_chunks // _RING
    n_ktiles = _DIM // 8
    n_tc = n_i // _CHUNK
    out_flat = jax.ShapeDtypeStruct((n_j * n_ktiles * n_tc * 8, _CHUNK),
                                    jnp.float32)
    mesh = plsc.VectorSubcoreMesh(core_axis_name="c", subcore_axis_name="s")

    @functools.partial(
        pl.kernel,
        mesh=mesh,
        out_type=(out_flat,) * 4,
        compiler_params=pltpu.CompilerParams(
            use_tc_tiling_on_sc=False, needs_layout_passes=False),
        scratch_types=(
            [pltpu.VMEM((n_chunks, _CHUNK), jnp.int32)]
            + [pltpu.VMEM((_CHUNK, _DIM), jnp.float32)] * (2 * _RING)
            + [pltpu.VMEM((_DIM, _CHUNK), jnp.float32)] * 8
            + [pltpu.VMEM((_DIM, _PITCH), jnp.float32)] * 2
            + [pltpu.SemaphoreType.DMA] * (_RING + 2)
        ),
    )
    def sc_kernel(xq, bt, et, ob, oe, od, oa, idx_v,
                  b0, b1, b2, b3, e0, e1, e2, e3,
                  sb0, se0, sd0, sa0, sb1, se1, sd1, sa1,
                  ptb, pte, g0, g1, g2, g3, w0, w1):
        brefs = (b0, b1, b2, b3)
        erefs = (e0, e1, e2, e3)
        gsem = (g0, g1, g2, g3)
        wsem = (w0, w1)
        stag = ((sb0, se0, sd0, sa0), (sb1, se1, sd1, sa1))
        c_ax = lax.axis_index("c")
        s_ax = lax.axis_index("s")
        wid = s_ax * nc + c_ax
        pltpu.sync_copy(xq.at[wid], idx_v)

        kvs = [lax.iota(jnp.int32, 16) + (h * _L) for h in range(_DIM // _L)]

        def issue_gather(cidx, slot):
            pltpu.async_copy(bt.at[idx_v.at[cidx]], brefs[slot], gsem[slot])
            pltpu.async_copy(et.at[idx_v.at[cidx]], erefs[slot], gsem[slot])

        for u in range(_PF):
            issue_gather(u, u)

        def group(gi, carry):
            for u in range(_RING):
                cc = gi * _RING + u
                s = u & 1
                sbs, ses, sds, sas = stag[s]
                bg = brefs[u]
                eg = erefs[u]

                pslot = (u + _PF) % _RING
                if u == 0:
                    issue_gather(cc + _PF, pslot)
                else:
                    @pl.when(gi < n_groups - 1)
                    def _pref(_cc=cc, _ps=pslot):
                        issue_gather(_cc + _PF, _ps)

                pltpu.make_async_copy(
                    bt.at[idx_v.at[cc]], bg, gsem[u]).wait()
                pltpu.make_async_copy(
                    et.at[idx_v.at[cc]], eg, gsem[u]).wait()

                def drain(_s=s):
                    for o_ref in stag[_s]:
                        for tr in range(n_ktiles):
                            pltpu.make_async_copy(
                                o_ref.at[pl.ds(tr * 8, 8)],
                                ob.at[pl.ds(0, 8)], wsem[_s]).wait()
                if u >= 2:
                    drain()
                else:
                    @pl.when(gi >= 1)
                    def _d():
                        drain()

                def rloop(rr, c2, _bg=bg, _eg=eg):
                    rvec = jnp.full((16,), rr, dtype=jnp.int32)
                    for h in range(_DIM // _L):
                        sl = pl.ds(h * _L, _L)
                        plsc.store_scatter(ptb, [kvs[h], rvec], _bg[rr, sl])
                        plsc.store_scatter(pte, [kvs[h], rvec], _eg[rr, sl])
                    return c2

                lax.fori_loop(0, _CHUNK, rloop, 0)

                def kloop(kk, c2, _sbs=sbs, _ses=ses, _sds=sds, _sas=sas):
                    for lg in range(8):
                        sl = pl.ds(lg * _L, _L)
                        bv = ptb[kk, sl]
                        ev = pte[kk, sl]
                        t = bv + ev
                        a0 = bv + 2.0 * ev
                        _sbs[kk, sl] = bv
                        _ses[kk, sl] = ev
                        _sds[kk, sl] = _mod1(t)
                        _sas[kk, sl] = _mod1(a0)
                    return c2

                lax.fori_loop(0, _DIM, kloop, 0)

                j = cc // tcw
                t = cc % tcw
                for tr in range(n_ktiles):
                    q0 = ((j * n_ktiles + tr) * n_tc + wid * tcw + t) * 8
                    src = pl.ds(tr * 8, 8)
                    dst = pl.ds(q0, 8)
                    pltpu.async_copy(sbs.at[src], ob.at[dst], wsem[s])
                    pltpu.async_copy(ses.at[src], oe.at[dst], wsem[s])
                    pltpu.async_copy(sds.at[src], od.at[dst], wsem[s])
                    pltpu.async_copy(sas.at[src], oa.at[dst], wsem[s])
            return carry

        lax.fori_loop(0, n_groups, group, 0)

        for s in range(2):
            for o_ref in stag[s]:
                for tr in range(n_ktiles):
                    pltpu.make_async_copy(
                        o_ref.at[pl.ds(tr * 8, 8)],
                        ob.at[pl.ds(0, 8)], wsem[s]).wait()

    return sc_kernel


def kernel(x, b_table, e_table):
    info = plsc.get_sparse_core_info()
    nw = info.num_cores * info.num_subcores
    nc = info.num_cores
    n_i, n_j = x.shape
    ipw = n_i // nw
    tcw = ipw // _CHUNK
    xq = (x.T.astype(jnp.int32)
          .reshape(n_j, nw, tcw, _CHUNK)
          .transpose(1, 0, 2, 3)
          .reshape(nw, n_j * tcw, _CHUNK))
    k = _make_sc_kernel(n_j, n_i, nw, nc)
    outs = k(xq, b_table, e_table)
    n_ktiles = _DIM // 8
    n_tc = n_i // _CHUNK
    res = []
    for o in outs:
        v5 = o.reshape(n_j, n_ktiles, n_tc, 8, _CHUNK)
        res.append(v5.transpose(2, 4, 0, 1, 3).reshape(n_i, n_j, _DIM))
    return tuple(res)


# speedup vs baseline: 2.8090x; 1.0786x over previous
import functools

import jax
import jax.numpy as jnp
from jax import lax
from jax.experimental import pallas as pl
from jax.experimental.pallas import tpu as pltpu
from jax.experimental.pallas import tpu_sc as plsc

_DIM = 32
_L = 16
_CHUNK = 128
_RING = 4
_PF = 3
_PITCH = _CHUNK + 1


def _mod1(v):
    r = lax.rem(v, 1.0)
    return jnp.where(r < 0.0, r + 1.0, r)


@functools.lru_cache(maxsize=None)
def _make_sc_kernel(n_j, n_i, nw, nc):
    ipw = n_i // nw
    tcw = ipw // _CHUNK
    n_chunks = n_j * tcw
    n_groups = n_chunks // _RING
    n_ktiles = _DIM // 8
    n_tc = n_i // _CHUNK
    out_flat = jax.ShapeDtypeStruct((n_j * n_ktiles * n_tc * 8, _CHUNK),
                                    jnp.float32)
    mesh = plsc.VectorSubcoreMesh(core_axis_name="c", subcore_axis_name="s")

    @functools.partial(
        pl.kernel,
        mesh=mesh,
        out_type=(out_flat,) * 4,
        compiler_params=pltpu.CompilerParams(
            use_tc_tiling_on_sc=False, needs_layout_passes=False),
        scratch_types=(
            [pltpu.VMEM((n_chunks, _CHUNK), jnp.int32)]
            + [pltpu.VMEM((_CHUNK, _DIM), jnp.float32)] * (2 * _RING)
            + [pltpu.VMEM((_DIM, _PITCH), jnp.float32)] * 8
            + [pltpu.SemaphoreType.DMA] * (_RING + 2)
        ),
    )
    def sc_kernel(xq, bt, et, ob, oe, od, oa, idx_v,
                  b0, b1, b2, b3, e0, e1, e2, e3,
                  sb0, se0, sd0, sa0, sb1, se1, sd1, sa1,
                  g0, g1, g2, g3, w0, w1):
        brefs = (b0, b1, b2, b3)
        erefs = (e0, e1, e2, e3)
        gsem = (g0, g1, g2, g3)
        wsem = (w0, w1)
        stag = ((sb0, se0, sd0, sa0), (sb1, se1, sd1, sa1))
        c_ax = lax.axis_index("c")
        s_ax = lax.axis_index("s")
        wid = s_ax * nc + c_ax
        pltpu.sync_copy(xq.at[wid], idx_v)

        kvs = [lax.iota(jnp.int32, 16) + (h * _L) for h in range(_DIM // _L)]

        def issue_gather(cidx, slot):
            pltpu.async_copy(bt.at[idx_v.at[cidx]], brefs[slot], gsem[slot])
            pltpu.async_copy(et.at[idx_v.at[cidx]], erefs[slot], gsem[slot])

        for u in range(_PF):
            issue_gather(u, u)

        def group(gi, carry):
            for u in range(_RING):
                cc = gi * _RING + u
                s = u & 1
                sbs, ses, sds, sas = stag[s]
                bg = brefs[u]
                eg = erefs[u]

                pslot = (u + _PF) % _RING
                if u == 0:
                    issue_gather(cc + _PF, pslot)
                else:
                    @pl.when(gi < n_groups - 1)
                    def _pref(_cc=cc, _ps=pslot):
                        issue_gather(_cc + _PF, _ps)

                pltpu.make_async_copy(
                    bt.at[idx_v.at[cc]], bg, gsem[u]).wait()
                pltpu.make_async_copy(
                    et.at[idx_v.at[cc]], eg, gsem[u]).wait()

                def drain(_s=s):
                    for o_ref in stag[_s]:
                        for tr in range(n_ktiles):
                            pltpu.make_async_copy(
                                o_ref.at[pl.ds(tr * 8, 8), pl.ds(0, _CHUNK)],
                                ob.at[pl.ds(0, 8)], wsem[_s]).wait()
                if u >= 2:
                    drain()
                else:
                    @pl.when(gi >= 1)
                    def _d():
                        drain()

                def rloop(rr, c2, _bg=bg, _eg=eg, _sbs=sbs, _ses=ses,
                          _sds=sds, _sas=sas):
                    rvec = jnp.full((16,), rr, dtype=jnp.int32)
                    for h in range(_DIM // _L):
                        sl = pl.ds(h * _L, _L)
                        bv = _bg[rr, sl]
                        ev = _eg[rr, sl]
                        t = bv + ev
                        a0 = bv + 2.0 * ev
                        plsc.store_scatter(_sbs, [kvs[h], rvec], bv)
                        plsc.store_scatter(_ses, [kvs[h], rvec], ev)
                        plsc.store_scatter(_sds, [kvs[h], rvec], _mod1(t))
                        plsc.store_scatter(_sas, [kvs[h], rvec], _mod1(a0))
                    return c2

                lax.fori_loop(0, _CHUNK, rloop, 0)

                j = cc // tcw
                t = cc % tcw
                for tr in range(n_ktiles):
                    q0 = ((j * n_ktiles + tr) * n_tc + wid * tcw + t) * 8
                    dst = pl.ds(q0, 8)
                    for src_ref, out_ref in ((sbs, ob), (ses, oe),
                                             (sds, od), (sas, oa)):
                        pltpu.async_copy(
                            src_ref.at[pl.ds(tr * 8, 8), pl.ds(0, _CHUNK)],
                            out_ref.at[dst], wsem[s])
            return carry

        lax.fori_loop(0, n_groups, group, 0)

        for s in range(2):
            for o_ref in stag[s]:
                for tr in range(n_ktiles):
                    pltpu.make_async_copy(
                        o_ref.at[pl.ds(tr * 8, 8), pl.ds(0, _CHUNK)],
                        ob.at[pl.ds(0, 8)], wsem[s]).wait()

    return sc_kernel


def kernel(x, b_table, e_table):
    info = plsc.get_sparse_core_info()
    nw = info.num_cores * info.num_subcores
    nc = info.num_cores
    n_i, n_j = x.shape
    ipw = n_i // nw
    tcw = ipw // _CHUNK
    xq = (x.T.astype(jnp.int32)
          .reshape(n_j, nw, tcw, _CHUNK)
          .transpose(1, 0, 2, 3)
          .reshape(nw, n_j * tcw, _CHUNK))
    k = _make_sc_kernel(n_j, n_i, nw, nc)
    outs = k(xq, b_table, e_table)
    n_ktiles = _DIM // 8
    n_tc = n_i // _CHUNK
    res = []
    for o in outs:
        v5 = o.reshape(n_j, n_ktiles, n_tc, 8, _CHUNK)
        res.append(v5.transpose(2, 4, 0, 1, 3).reshape(n_i, n_j, _DIM))
    return tuple(res)


# speedup vs baseline: 2.8489x; 1.0142x over previous
import functools

import jax
import jax.numpy as jnp
from jax import lax
from jax.experimental import pallas as pl
from jax.experimental.pallas import tpu as pltpu
from jax.experimental.pallas import tpu_sc as plsc

_DIM = 32
_L = 16
_CHUNK = 128
_RING = 4
_PF = 3
_PITCH = _CHUNK + 1
_RUR = 4


def _mod1(v):
    r = lax.rem(v, 1.0)
    return jnp.where(r < 0.0, r + 1.0, r)


@functools.lru_cache(maxsize=None)
def _make_sc_kernel(n_j, n_i, nw, nc):
    ipw = n_i // nw
    tcw = ipw // _CHUNK
    n_chunks = n_j * tcw
    n_groups = n_chunks // _RING
    n_ktiles = _DIM // 8
    n_tc = n_i // _CHUNK
    out_flat = jax.ShapeDtypeStruct((n_j * n_ktiles * n_tc * 8, _CHUNK),
                                    jnp.float32)
    mesh = plsc.VectorSubcoreMesh(core_axis_name="c", subcore_axis_name="s")

    @functools.partial(
        pl.kernel,
        mesh=mesh,
        out_type=(out_flat,) * 4,
        compiler_params=pltpu.CompilerParams(
            use_tc_tiling_on_sc=False, needs_layout_passes=False),
        scratch_types=(
            [pltpu.VMEM((n_chunks, _CHUNK), jnp.int32)]
            + [pltpu.VMEM((_CHUNK, _DIM), jnp.float32)] * (2 * _RING)
            + [pltpu.VMEM((_DIM, _PITCH), jnp.float32)] * 8
            + [pltpu.SemaphoreType.DMA] * (_RING + 2)
        ),
    )
    def sc_kernel(xq, bt, et, ob, oe, od, oa, idx_v,
                  b0, b1, b2, b3, e0, e1, e2, e3,
                  sb0, se0, sd0, sa0, sb1, se1, sd1, sa1,
                  g0, g1, g2, g3, w0, w1):
        brefs = (b0, b1, b2, b3)
        erefs = (e0, e1, e2, e3)
        gsem = (g0, g1, g2, g3)
        wsem = (w0, w1)
        stag = ((sb0, se0, sd0, sa0), (sb1, se1, sd1, sa1))
        c_ax = lax.axis_index("c")
        s_ax = lax.axis_index("s")
        wid = s_ax * nc + c_ax
        pltpu.sync_copy(xq.at[wid], idx_v)

        kvs = [lax.iota(jnp.int32, 16) + (h * _L) for h in range(_DIM // _L)]

        def issue_gather(cidx, slot):
            pltpu.async_copy(bt.at[idx_v.at[cidx]], brefs[slot], gsem[slot])
            pltpu.async_copy(et.at[idx_v.at[cidx]], erefs[slot], gsem[slot])

        for u in range(_PF):
            issue_gather(u, u)

        def group(gi, carry):
            for u in range(_RING):
                cc = gi * _RING + u
                s = u & 1
                sbs, ses, sds, sas = stag[s]
                bg = brefs[u]
                eg = erefs[u]

                pslot = (u + _PF) % _RING
                if u == 0:
                    issue_gather(cc + _PF, pslot)
                else:
                    @pl.when(gi < n_groups - 1)
                    def _pref(_cc=cc, _ps=pslot):
                        issue_gather(_cc + _PF, _ps)

                pltpu.make_async_copy(
                    bt.at[idx_v.at[cc]], bg, gsem[u]).wait()
                pltpu.make_async_copy(
                    et.at[idx_v.at[cc]], eg, gsem[u]).wait()

                def drain(_s=s):
                    for o_ref in stag[_s]:
                        for tr in range(n_ktiles):
                            pltpu.make_async_copy(
                                o_ref.at[pl.ds(tr * 8, 8), pl.ds(0, _CHUNK)],
                                ob.at[pl.ds(0, 8)], wsem[_s]).wait()
                if u >= 2:
                    drain()
                else:
                    @pl.when(gi >= 1)
                    def _d():
                        drain()

                def rloop(r4, c2, _bg=bg, _eg=eg, _sbs=sbs, _ses=ses,
                          _sds=sds, _sas=sas):
                    r0 = r4 * _RUR
                    for ur in range(_RUR):
                        rr = r0 + ur
                        rvec = jnp.full((16,), rr, dtype=jnp.int32)
                        for h in range(_DIM // _L):
                            sl = pl.ds(h * _L, _L)
                            bv = _bg[rr, sl]
                            ev = _eg[rr, sl]
                            t = bv + ev
                            a0 = bv + 2.0 * ev
                            plsc.store_scatter(_sbs, [kvs[h], rvec], bv)
                            plsc.store_scatter(_ses, [kvs[h], rvec], ev)
                            plsc.store_scatter(_sds, [kvs[h], rvec], _mod1(t))
                            plsc.store_scatter(_sas, [kvs[h], rvec],
                                               _mod1(a0))
                    return c2

                lax.fori_loop(0, _CHUNK // _RUR, rloop, 0)

                j = cc // tcw
                t = cc % tcw
                for tr in range(n_ktiles):
                    q0 = ((j * n_ktiles + tr) * n_tc + wid * tcw + t) * 8
                    dst = pl.ds(q0, 8)
                    for src_ref, out_ref in ((sbs, ob), (ses, oe),
                                             (sds, od), (sas, oa)):
                        pltpu.async_copy(
                            src_ref.at[pl.ds(tr * 8, 8), pl.ds(0, _CHUNK)],
                            out_ref.at[dst], wsem[s])
            return carry

        lax.fori_loop(0, n_groups, group, 0)

        for s in range(2):
            for o_ref in stag[s]:
                for tr in range(n_ktiles):
                    pltpu.make_async_copy(
                        o_ref.at[pl.ds(tr * 8, 8), pl.ds(0, _CHUNK)],
                        ob.at[pl.ds(0, 8)], wsem[s]).wait()

    return sc_kernel


def kernel(x, b_table, e_table):
    info = plsc.get_sparse_core_info()
    nw = info.num_cores * info.num_subcores
    nc = info.num_cores
    n_i, n_j = x.shape
    ipw = n_i // nw
    tcw = ipw // _CHUNK
    xq = (x.T.astype(jnp.int32)
          .reshape(n_j, nw, tcw, _CHUNK)
          .transpose(1, 0, 2, 3)
          .reshape(nw, n_j * tcw, _CHUNK))
    k = _make_sc_kernel(n_j, n_i, nw, nc)
    outs = k(xq, b_table, e_table)
    n_ktiles = _DIM // 8
    n_tc = n_i // _CHUNK
    res = []
    for o in outs:
        v5 = o.reshape(n_j, n_ktiles, n_tc, 8, _CHUNK)
        res.append(v5.transpose(2, 4, 0, 1, 3).reshape(n_i, n_j, _DIM))
    return tuple(res)
